# baseline (speedup 1.0000x reference)
import functools

import jax
import jax.numpy as jnp
from jax import lax
from jax.experimental import pallas as pl
from jax.experimental.pallas import tpu as pltpu
from jax.experimental.pallas import tpu_sc as plsc

_NW = 32
_CH = 128
_PAIR = 2
_NPB = 3


def kernel(x, embedding):
    B, S = x.shape
    V, D = embedding.shape
    total = B * S
    bpw = total // _NW
    nch = bpw // _CH
    npair = nch // _PAIR
    ngroups = npair // _NPB
    rows_pp = _PAIR * _CH
    assert bpw * _NW == total and nch * _CH == bpw and npair * _PAIR == nch
    assert B % rows_pp == 0

    idx = x.T.reshape(_NW, nch, _CH)
    mesh = plsc.VectorSubcoreMesh(core_axis_name="c", subcore_axis_name="s")

    @functools.partial(
        pl.kernel,
        out_type=jax.ShapeDtypeStruct((S, B, D), jnp.float32),
        mesh=mesh,
        scratch_types=[
            pltpu.VMEM((nch, _CH), jnp.int32),
            pltpu.VMEM((_NPB, rows_pp, D), jnp.float32),
        ]
        + [pltpu.SemaphoreType.DMA] * (2 * _NPB),
    )
    def emb_lookup(table_hbm, idx_hbm, out_hbm, idx_v, rows_v, *sems):
        gsem, wsem = sems[:_NPB], sems[_NPB:]
        wid = lax.axis_index("s") * 2 + lax.axis_index("c")
        pltpu.sync_copy(idx_hbm.at[wid], idx_v)

        def start_gather_pair(p, b):
            for h in range(_PAIR):
                pltpu.async_copy(
                    table_hbm.at[idx_v.at[p * _PAIR + h]],
                    rows_v.at[b, pl.ds(h * _CH, _CH)],
                    gsem[b],
                )

        def wait_gather_pair(p, b):
            for h in range(_PAIR):
                pltpu.make_async_copy(
                    table_hbm.at[idx_v.at[p * _PAIR + h]],
                    rows_v.at[b, pl.ds(h * _CH, _CH)],
                    gsem[b],
                ).wait()

        def out_slice(p):
            flat = wid * bpw + p * rows_pp
            return out_hbm.at[flat // B, pl.ds(flat % B, rows_pp)]

        for b in range(_NPB):
            start_gather_pair(b, b)

        @pl.loop(0, ngroups * _NPB, step=_NPB)
        def _group(p0):
            for b in range(_NPB):
                p = p0 + b
                wait_gather_pair(p, b)
                pltpu.async_copy(rows_v.at[b], out_slice(p), wsem[b])

                @pl.when(p + _NPB < npair)
                def _refill():
                    pltpu.make_async_copy(rows_v.at[b], out_slice(p), wsem[b]).wait()
                    start_gather_pair(p + _NPB, b)

        for p in range(ngroups * _NPB, npair):
            b = p % _NPB
            wait_gather_pair(p, b)
            pltpu.async_copy(rows_v.at[b], out_slice(p), wsem[b])

        for b in range(_NPB):
            pltpu.make_async_copy(rows_v.at[b], out_slice(0), wsem[b]).wait()

    out = emb_lookup(embedding, idx)
    return out.transpose(1, 0, 2)


# speedup vs baseline: 1.0087x; 1.0087x over previous
import functools

import jax
import jax.numpy as jnp
from jax import lax
from jax.experimental import pallas as pl
from jax.experimental.pallas import tpu as pltpu
from jax.experimental.pallas import tpu_sc as plsc

_NW = 32
_CH = 128
_NBUF = 7


def kernel(x, embedding):
    B, S = x.shape
    V, D = embedding.shape
    total = B * S
    bpw = total // _NW
    nch = bpw // _CH
    assert bpw * _NW == total and nch * _CH == bpw
    assert B % _CH == 0
    ngroups = nch // _NBUF

    idx = x.T.reshape(_NW, nch, _CH)
    mesh = plsc.VectorSubcoreMesh(core_axis_name="c", subcore_axis_name="s")

    @functools.partial(
        pl.kernel,
        out_type=jax.ShapeDtypeStruct((S, B, D), jnp.float32),
        mesh=mesh,
        scratch_types=[
            pltpu.VMEM((nch, _CH), jnp.int32),
            pltpu.VMEM((_NBUF, _CH, D), jnp.float32),
        ]
        + [pltpu.SemaphoreType.DMA] * (2 * _NBUF),
    )
    def emb_lookup(table_hbm, idx_hbm, out_hbm, idx_v, rows_v, *sems):
        gsem, wsem = sems[:_NBUF], sems[_NBUF:]
        wid = lax.axis_index("s") * 2 + lax.axis_index("c")
        pltpu.sync_copy(idx_hbm.at[wid], idx_v)

        def start_gather(j, b):
            pltpu.async_copy(table_hbm.at[idx_v.at[j]], rows_v.at[b], gsem[b])

        def out_slice(j):
            flat = wid * bpw + j * _CH
            return out_hbm.at[flat // B, pl.ds(flat % B, _CH)]

        for b in range(_NBUF):
            start_gather(b, b)

        @pl.loop(0, ngroups * _NBUF, step=_NBUF)
        def _group(j0):
            for b in range(_NBUF):
                j = j0 + b
                pltpu.make_async_copy(
                    table_hbm.at[idx_v.at[j]], rows_v.at[b], gsem[b]
                ).wait()
                pltpu.async_copy(rows_v.at[b], out_slice(j), wsem[b])

                @pl.when(j + _NBUF < nch)
                def _refill():
                    pltpu.make_async_copy(
                        rows_v.at[b], out_slice(j), wsem[b]
                    ).wait()
                    start_gather(j + _NBUF, b)

        for j in range(ngroups * _NBUF, nch):
            b = j % _NBUF
            pltpu.make_async_copy(
                table_hbm.at[idx_v.at[j]], rows_v.at[b], gsem[b]
            ).wait()
            pltpu.async_copy(rows_v.at[b], out_slice(j), wsem[b])

        for b in range(_NBUF):
            pltpu.make_async_copy(rows_v.at[b], out_slice(0), wsem[b]).wait()

    out = emb_lookup(embedding, idx)
    return out.transpose(1, 0, 2)


# speedup vs baseline: 1.0094x; 1.0007x over previous
import functools

import jax
import jax.numpy as jnp
from jax import lax
from jax.experimental import pallas as pl
from jax.experimental.pallas import tpu as pltpu
from jax.experimental.pallas import tpu_sc as plsc

_NW = 32
_CH = 128
_NBUF = 7


def kernel(x, embedding):
    B, S = x.shape
    V, D = embedding.shape
    total = B * S
    bpw = total // _NW
    nch = bpw // _CH
    assert bpw * _NW == total and nch * _CH == bpw
    assert B % _CH == 0
    ngroups = nch // _NBUF

    idx = x.T.reshape(_NW, nch, _CH).astype(jnp.int32)
    mesh = plsc.VectorSubcoreMesh(core_axis_name="c", subcore_axis_name="s")

    @functools.partial(
        pl.kernel,
        out_type=jax.ShapeDtypeStruct((S, B, D), jnp.float32),
        mesh=mesh,
        scratch_types=[
            pltpu.VMEM((nch, _CH), jnp.int32),
            pltpu.VMEM((_NBUF, _CH, D), jnp.float32),
        ]
        + [pltpu.SemaphoreType.DMA] * (2 * _NBUF),
    )
    def emb_lookup(table_hbm, idx_hbm, out_hbm, idx_v, rows_v, *sems):
        gsem, wsem = sems[:_NBUF], sems[_NBUF:]
        wid = lax.axis_index("s") * 2 + lax.axis_index("c")
        pltpu.sync_copy(idx_hbm.at[wid], idx_v)

        def start_gather(j, b):
            pltpu.async_copy(table_hbm.at[idx_v.at[j]], rows_v.at[b], gsem[b])

        def out_slice(j):
            flat = wid * bpw + j * _CH
            return out_hbm.at[flat // B, pl.ds(flat % B, _CH)]

        for b in range(_NBUF):
            start_gather(b, b)

        @pl.loop(0, ngroups * _NBUF, step=_NBUF)
        def _group(j0):
            for b in range(_NBUF):
                j = j0 + b
                pltpu.make_async_copy(
                    table_hbm.at[idx_v.at[j]], rows_v.at[b], gsem[b]
                ).wait()
                pltpu.async_copy(rows_v.at[b], out_slice(j), wsem[b])

                @pl.when(j + _NBUF < nch)
                def _refill():
                    pltpu.make_async_copy(
                        rows_v.at[b], out_slice(j), wsem[b]
                    ).wait()
                    start_gather(j + _NBUF, b)

        for j in range(ngroups * _NBUF, nch):
            b = j % _NBUF
            pltpu.make_async_copy(
                table_hbm.at[idx_v.at[j]], rows_v.at[b], gsem[b]
            ).wait()
            pltpu.async_copy(rows_v.at[b], out_slice(j), wsem[b])

        for b in range(_NBUF):
            pltpu.make_async_copy(rows_v.at[b], out_slice(0), wsem[b]).wait()

    out = emb_lookup(embedding, idx)
    return out.transpose(1, 0, 2)
